# Initial kernel scaffold; baseline (speedup 1.0000x reference)
import functools

import jax
import jax.numpy as jnp
from jax import lax
from jax.experimental import pallas as pl
from jax.experimental.pallas import tpu as pltpu
from jax.experimental.pallas import tpu_sc as plsc

NC = 2
NS = 16
NW = NC * NS

B = 16384 * 26
D = 32
BPW = B // NW
CH = 128
NCH = BPW // CH
NBUF = 2

_mesh = plsc.VectorSubcoreMesh(core_axis_name="c", subcore_axis_name="s")


@functools.partial(
    pl.kernel,
    out_type=jax.ShapeDtypeStruct((B, D), jnp.float32),
    mesh=_mesh,
    scratch_types=[
        pltpu.VMEM((NCH, CH), jnp.int32),
        pltpu.VMEM((NBUF, CH, D), jnp.float32),
        pltpu.SemaphoreType.DMA,
        pltpu.SemaphoreType.DMA,
    ],
)
def _gather_kernel(idx_hbm, table_hbm, out_hbm, idx_v, rows_v, gsem, wsem):
    wid = lax.axis_index("s") * NC + lax.axis_index("c")
    pltpu.sync_copy(idx_hbm.at[pl.ds(wid * NCH, NCH)], idx_v)
    out_base = wid * BPW

    for b in range(NBUF):
        pltpu.async_copy(table_hbm.at[idx_v.at[b]], rows_v.at[b], gsem)

    def outer(g, _):
        for b in range(NBUF):
            j = g + b

            @pl.when(j >= NBUF)
            def _start_next():
                pltpu.make_async_copy(
                    rows_v.at[b], out_hbm.at[pl.ds(out_base, CH)], wsem
                ).wait()
                pltpu.async_copy(table_hbm.at[idx_v.at[j]], rows_v.at[b], gsem)

            pltpu.make_async_copy(
                table_hbm.at[idx_v.at[b]], rows_v.at[b], gsem
            ).wait()
            pltpu.async_copy(
                rows_v.at[b], out_hbm.at[pl.ds(out_base + j * CH, CH)], wsem
            )
        return _

    pl.loop(0, NCH, step=NBUF, unroll=1)(outer)(None)

    for b in range(NBUF):
        pltpu.make_async_copy(
            rows_v.at[b], out_hbm.at[pl.ds(out_base, CH)], wsem
        ).wait()


def kernel(indices, parameter):
    idx_flat = indices.reshape(NW * NCH, CH)
    out = _gather_kernel(idx_flat, parameter)
    return out.reshape(indices.shape[0], indices.shape[1], D)


# speedup vs baseline: 1.4692x; 1.4692x over previous
import functools

import jax
import jax.numpy as jnp
from jax import lax
from jax.experimental import pallas as pl
from jax.experimental.pallas import tpu as pltpu
from jax.experimental.pallas import tpu_sc as plsc

NC = 2
NS = 16
NW = NC * NS

B = 16384 * 26
D = 32
BPW = B // NW
CH = 128
NCH = BPW // CH
NBUF = 2


@functools.cache
def _build_gather_kernel():
    mesh = plsc.VectorSubcoreMesh(core_axis_name="c", subcore_axis_name="s")

    @functools.partial(
        pl.kernel,
        out_type=jax.ShapeDtypeStruct((B, D), jnp.float32),
        mesh=mesh,
        compiler_params=pltpu.CompilerParams(use_tc_tiling_on_sc=False),
        scratch_types=[
            pltpu.VMEM((NCH, CH), jnp.int32),
            pltpu.VMEM((NBUF, CH, D), jnp.float32),
            pltpu.SemaphoreType.DMA,
            pltpu.SemaphoreType.DMA,
        ],
    )
    def gather_kernel(idx_hbm, table_hbm, out_hbm, idx_v, rows_v, gsem, wsem):
        wid = lax.axis_index("s") * NC + lax.axis_index("c")
        pltpu.sync_copy(idx_hbm.at[pl.ds(wid * NCH, NCH)], idx_v)
        out_base = wid * BPW

        for b in range(NBUF):
            pltpu.async_copy(table_hbm.at[idx_v.at[b]], rows_v.at[b], gsem)

        def outer(g):
            for b in range(NBUF):
                j = g + b

                @pl.when(j >= NBUF)
                def _start_next():
                    pltpu.make_async_copy(
                        rows_v.at[b], out_hbm.at[pl.ds(out_base, CH)], wsem
                    ).wait()
                    pltpu.async_copy(
                        table_hbm.at[idx_v.at[j]], rows_v.at[b], gsem
                    )

                pltpu.make_async_copy(
                    table_hbm.at[idx_v.at[b]], rows_v.at[b], gsem
                ).wait()
                pltpu.async_copy(
                    rows_v.at[b], out_hbm.at[pl.ds(out_base + j * CH, CH)], wsem
                )

        pl.loop(0, NCH, step=NBUF, unroll=1)(outer)

        for b in range(NBUF):
            pltpu.make_async_copy(
                rows_v.at[b], out_hbm.at[pl.ds(out_base, CH)], wsem
            ).wait()

    return gather_kernel


def kernel(indices, parameter):
    idx_flat = indices.reshape(NW * NCH, CH)
    out = _build_gather_kernel()(idx_flat, parameter)
    return out.reshape(indices.shape[0], indices.shape[1], D)


# speedup vs baseline: 1.5751x; 1.0721x over previous
import functools

import jax
import jax.numpy as jnp
from jax import lax
from jax.experimental import pallas as pl
from jax.experimental.pallas import tpu as pltpu
from jax.experimental.pallas import tpu_sc as plsc

NC = 2
NS = 16
NW = NC * NS

B = 16384 * 26
D = 32
BPW = B // NW
CH = 128
NCH = BPW // CH
G = 8
NG = NCH // G
GR = G * CH


@functools.cache
def _build_gather_kernel():
    mesh = plsc.VectorSubcoreMesh(core_axis_name="c", subcore_axis_name="s")

    @functools.partial(
        pl.kernel,
        out_type=jax.ShapeDtypeStruct((B, D), jnp.float32),
        mesh=mesh,
        compiler_params=pltpu.CompilerParams(use_tc_tiling_on_sc=False),
        scratch_types=[
            pltpu.VMEM((NCH, CH), jnp.int32),
            pltpu.VMEM((2, GR, D), jnp.float32),
            pltpu.SemaphoreType.DMA,
            pltpu.SemaphoreType.DMA,
            pltpu.SemaphoreType.DMA,
            pltpu.SemaphoreType.DMA,
        ],
    )
    def gather_kernel(idx_hbm, table_hbm, out_hbm, idx_v, rows_v,
                      gsem0, gsem1, wsem0, wsem1):
        gsems = (gsem0, gsem1)
        wsems = (wsem0, wsem1)
        wid = lax.axis_index("s") * NC + lax.axis_index("c")
        pltpu.sync_copy(idx_hbm.at[pl.ds(wid * NCH, NCH)], idx_v)
        out_base = wid * BPW

        def fire(g, buf):
            for k in range(G):
                pltpu.async_copy(
                    table_hbm.at[idx_v.at[g * G + k]],
                    rows_v.at[buf].at[pl.ds(k * CH, CH)],
                    gsems[buf],
                )

        def drain_gathers(buf):
            pltpu.make_async_copy(
                table_hbm.at[pl.ds(0, GR)], rows_v.at[buf], gsems[buf]
            ).wait()

        def start_write(g, buf):
            pltpu.async_copy(
                rows_v.at[buf], out_hbm.at[pl.ds(out_base + g * GR, GR)],
                wsems[buf],
            )

        def wait_write(buf):
            pltpu.make_async_copy(
                rows_v.at[buf], out_hbm.at[pl.ds(out_base, GR)], wsems[buf]
            ).wait()

        def step(g, buf, first=False, last=False):
            if not last:
                if not first:
                    wait_write(1 - buf)
                fire(g + 1, 1 - buf)
            drain_gathers(buf)
            start_write(g, buf)

        fire(0, 0)
        step(0, 0, first=True)
        pl.loop(1, NG - 2, step=2, unroll=1)(
            lambda g: (step(g, 1), step(g + 1, 0)) and None
        )
        step(NG - 2, 1)
        step(NG - 1, 0, last=True)

        wait_write(0)
        wait_write(1)

    return gather_kernel


def kernel(indices, parameter):
    idx_flat = indices.reshape(NW * NCH, CH)
    out = _build_gather_kernel()(idx_flat, parameter)
    return out.reshape(indices.shape[0], indices.shape[1], D)


# speedup vs baseline: 1.9379x; 1.2303x over previous
import functools

import jax
import jax.numpy as jnp
from jax import lax
from jax.experimental import pallas as pl
from jax.experimental.pallas import tpu as pltpu
from jax.experimental.pallas import tpu_sc as plsc

NC = 2
NS = 16
NW = NC * NS

NB = 16384
F = 26
D = 32
IPW = NB // NW
IG = 8
NG = IPW // IG


@functools.cache
def _build_gather_kernel():
    mesh = plsc.VectorSubcoreMesh(core_axis_name="c", subcore_axis_name="s")

    @functools.partial(
        pl.kernel,
        out_type=jax.ShapeDtypeStruct((NB, 32, 128), jnp.float32),
        mesh=mesh,
        compiler_params=pltpu.CompilerParams(use_tc_tiling_on_sc=False),
        scratch_types=[
            pltpu.VMEM((IPW, F), jnp.int32),
            pltpu.VMEM((2, IG, F, D), jnp.float32),
            pltpu.SemaphoreType.DMA,
            pltpu.SemaphoreType.DMA,
            pltpu.SemaphoreType.DMA,
            pltpu.SemaphoreType.DMA,
        ],
    )
    def gather_kernel(idx_hbm, table_hbm, out_hbm, idx_v, rows_v,
                      gsem0, gsem1, wsem0, wsem1):
        gsems = (gsem0, gsem1)
        wsems = (wsem0, wsem1)
        wid = lax.axis_index("s") * NC + lax.axis_index("c")
        pltpu.sync_copy(idx_hbm.at[pl.ds(wid * IPW, IPW)], idx_v)
        i_base0 = wid * IPW

        def fire(g, buf):
            for l in range(IG):
                pltpu.async_copy(
                    table_hbm.at[idx_v.at[g * IG + l]],
                    rows_v.at[buf].at[l],
                    gsems[buf],
                )

        def drain_gathers(buf):
            pltpu.make_async_copy(
                out_hbm.at[pl.ds(0, IG), pl.ds(0, F), pl.ds(0, D)],
                rows_v.at[buf],
                gsems[buf],
            ).wait()

        def start_write(g, buf):
            pltpu.async_copy(
                rows_v.at[buf],
                out_hbm.at[pl.ds(i_base0 + g * IG, IG), pl.ds(0, F),
                           pl.ds(0, D)],
                wsems[buf],
            )

        def wait_write(buf):
            pltpu.make_async_copy(
                rows_v.at[buf],
                out_hbm.at[pl.ds(0, IG), pl.ds(0, F), pl.ds(0, D)],
                wsems[buf],
            ).wait()

        def step(g, buf, first=False, last=False):
            if not last:
                if not first:
                    wait_write(1 - buf)
                fire(g + 1, 1 - buf)
            drain_gathers(buf)
            start_write(g, buf)

        fire(0, 0)
        step(0, 0, first=True)
        pl.loop(1, NG - 1, step=2, unroll=1)(
            lambda g: (step(g, 1), step(g + 1, 0)) and None
        )
        step(NG - 1, 1, last=True)

        wait_write(0)
        wait_write(1)

    return gather_kernel


def kernel(indices, parameter):
    out = _build_gather_kernel()(indices, parameter)
    return out[:, :F, :D]
